# baseline (speedup 1.0000x reference)
import jax
import jax.numpy as jnp
from jax.experimental import pallas as pl

_BB = 32


def _body(x_ref, m_ref, o_ref):
    x = x_ref[...]
    m = m_ref[...].astype(x.dtype)
    s = jnp.sum(x * m[:, :, None], axis=1)
    c = jnp.sum(m, axis=1, keepdims=True)
    o_ref[...] = s / c


def kernel(inputs, mask):
    B, L, D = inputs.shape
    grid = (B // _BB,)
    return pl.pallas_call(
        _body,
        grid=grid,
        in_specs=[
            pl.BlockSpec((_BB, L, D), lambda i: (i, 0, 0)),
            pl.BlockSpec((_BB, L), lambda i: (i, 0)),
        ],
        out_specs=pl.BlockSpec((_BB, D), lambda i: (i, 0)),
        out_shape=jax.ShapeDtypeStruct((B, D), inputs.dtype),
    )(inputs, mask)


# speedup vs baseline: 1.3747x; 1.3747x over previous
import jax
import jax.numpy as jnp
from jax.experimental import pallas as pl

_BB = 128


def _body(x_ref, m_ref, o_ref):
    x = x_ref[...]
    m = m_ref[...].astype(x.dtype)
    s = jnp.sum(x * m[:, :, None], axis=1)
    c = jnp.sum(m, axis=1, keepdims=True)
    o_ref[...] = s / c


def kernel(inputs, mask):
    B, L, D = inputs.shape
    grid = (B // _BB,)
    return pl.pallas_call(
        _body,
        grid=grid,
        in_specs=[
            pl.BlockSpec((_BB, L, D), lambda i: (i, 0, 0)),
            pl.BlockSpec((_BB, L), lambda i: (i, 0)),
        ],
        out_specs=pl.BlockSpec((_BB, D), lambda i: (i, 0)),
        out_shape=jax.ShapeDtypeStruct((B, D), inputs.dtype),
    )(inputs, mask.astype(jnp.uint8))


# speedup vs baseline: 1.3772x; 1.0018x over previous
import jax
import jax.numpy as jnp
from jax.experimental import pallas as pl

_BB = 128


def _body(x_ref, m_ref, o_ref):
    x = x_ref[...]
    m = m_ref[...].astype(x.dtype)
    s = jnp.sum(x * m[:, :, None], axis=1)
    c = jnp.sum(m, axis=1, keepdims=True)
    o_ref[...] = s / c


def kernel(inputs, mask):
    B, L, D = inputs.shape
    grid = (B // _BB,)
    return pl.pallas_call(
        _body,
        grid=grid,
        in_specs=[
            pl.BlockSpec((_BB, L, D), lambda i: (i, 0, 0)),
            pl.BlockSpec((_BB, L), lambda i: (i, 0)),
        ],
        out_specs=pl.BlockSpec((_BB, D), lambda i: (i, 0)),
        out_shape=jax.ShapeDtypeStruct((B, D), inputs.dtype),
    )(inputs, mask.view(jnp.uint8))
